# baseline (speedup 1.0000x reference)
import functools

import jax
import jax.numpy as jnp
from jax.experimental import pallas as pl
from jax.experimental.pallas import tpu as pltpu

BM = 256
BF = 1024


def _tc_moe_kernel(nfb, e_arr, xi_arr, val_arr, x_ref, gate_ref, up_ref,
                   down_ref, w_ref, out_ref):
    j = pl.program_id(0)
    i = pl.program_id(1)

    @pl.when(val_arr[i] == 1)
    def _():
        rows = pl.ds(i * BM, BM)
        xb = x_ref[...].astype(jnp.bfloat16)
        g = jnp.dot(xb, gate_ref[0].astype(jnp.bfloat16),
                    preferred_element_type=jnp.float32)
        u = jnp.dot(xb, up_ref[0].astype(jnp.bfloat16),
                    preferred_element_type=jnp.float32)
        h = (g * jax.nn.sigmoid(g) * u).astype(jnp.bfloat16)
        c = jnp.dot(h, down_ref[0].astype(jnp.bfloat16),
                    preferred_element_type=jnp.float32)

        @pl.when(j == 0)
        def _():
            out_ref[rows, :] = c

        @pl.when(jnp.logical_and(j > 0, j < nfb - 1))
        def _():
            out_ref[rows, :] += c

        @pl.when(jnp.logical_and(j == nfb - 1, nfb > 1))
        def _():
            out_ref[rows, :] = (out_ref[rows, :] + c) * w_ref[rows, :]


def _grouped_mlp(x_pad, gate_up_w, down_w, w_pad, e_arr, xi_arr, val_arr):
    m_pad, d = x_pad.shape
    e, _, f2 = gate_up_w.shape
    f = f2 // 2
    nfb = f // BF
    nt = m_pad // BM

    def full(j, i, ea, xa, va):
        return (0, 0)

    def xmap(j, i, ea, xa, va):
        return (xa[i], 0)

    def gmap(j, i, ea, xa, va):
        return (ea[i], 0, j)

    def umap(j, i, ea, xa, va):
        return (ea[i], 0, nfb + j)

    def dmap(j, i, ea, xa, va):
        return (ea[i], j, 0)

    grid_spec = pltpu.PrefetchScalarGridSpec(
        num_scalar_prefetch=3,
        grid=(nfb, nt),
        in_specs=[
            pl.BlockSpec((BM, d), xmap),
            pl.BlockSpec((1, d, BF), gmap),
            pl.BlockSpec((1, d, BF), umap),
            pl.BlockSpec((1, BF, d), dmap),
            pl.BlockSpec((m_pad, 1), full),
        ],
        out_specs=pl.BlockSpec((m_pad, d), full),
    )
    return pl.pallas_call(
        functools.partial(_tc_moe_kernel, nfb),
        grid_spec=grid_spec,
        out_shape=jax.ShapeDtypeStruct((m_pad, d), jnp.float32),
        compiler_params=pltpu.CompilerParams(
            dimension_semantics=("arbitrary", "arbitrary")),
    )(e_arr, xi_arr, val_arr, x_pad, gate_up_w, gate_up_w, down_w,
      w_pad.reshape(m_pad, 1))


def kernel(x, gate_w, gate_up_w, down_w):
    b, s, d = x.shape
    e, _, f2 = gate_up_w.shape
    topk = 2
    x_flat = x.reshape(-1, d)
    t = x_flat.shape[0]
    n_pairs = t * topk
    m_pad = n_pairs + (e - 1) * BM
    nt = m_pad // BM

    logits = x_flat @ gate_w.T
    m1 = jnp.max(logits, axis=-1)
    a1 = jnp.argmax(logits, axis=-1).astype(jnp.int32)
    eids = jnp.arange(e, dtype=jnp.int32)
    masked = jnp.where(eids[None, :] == a1[:, None], -jnp.inf, logits)
    m2 = jnp.max(masked, axis=-1)
    a2 = jnp.argmax(masked, axis=-1).astype(jnp.int32)
    denom = jnp.sum(jnp.exp(logits - m1[:, None]), axis=-1)
    p1 = 1.0 / denom
    p2 = jnp.exp(m2 - m1) / denom
    e_flat = jnp.stack([a1, a2], axis=-1).reshape(-1)
    w_flat = jnp.stack([p1, p2], axis=-1).reshape(-1)

    oh = (e_flat[:, None] == eids[None, :]).astype(jnp.int32)
    cum = jnp.cumsum(oh, axis=0)
    rank = jnp.take_along_axis(cum, e_flat[:, None], axis=1)[:, 0] - 1
    counts = cum[-1]
    pad_counts = ((counts + BM - 1) // BM) * BM
    pad_off = jnp.concatenate([jnp.zeros((1,), jnp.int32),
                               jnp.cumsum(pad_counts).astype(jnp.int32)])
    dest = pad_off[e_flat] + rank
    tok_flat = jnp.arange(n_pairs, dtype=jnp.int32) // topk
    src = jnp.zeros((m_pad,), jnp.int32).at[dest].set(tok_flat)
    w_pad = jnp.zeros((m_pad,), jnp.float32).at[dest].set(w_flat)

    ti = jnp.arange(nt, dtype=jnp.int32)
    tile_e = (jnp.sum((ti[:, None] * BM >= pad_off[None, 1:]).astype(
        jnp.int32), axis=1)).astype(jnp.int32)
    valid = (ti * BM < pad_off[e]).astype(jnp.int32)
    i_last = (pad_off[e] // BM - 1).astype(jnp.int32)
    e_last = jnp.clip(tile_e[i_last], 0, e - 1)
    e_arr = jnp.where(valid == 1, jnp.clip(tile_e, 0, e - 1), e_last)
    xi_arr = jnp.where(valid == 1, ti, i_last)

    x_pad = x_flat[src]
    y_pad = _grouped_mlp(x_pad, gate_up_w, down_w, w_pad, e_arr, xi_arr,
                         valid)
    d2 = dest.reshape(t, topk)
    out_flat = y_pad[d2[:, 0]] + y_pad[d2[:, 1]]
    return out_flat.reshape(b, s, d)
